# baseline (speedup 1.0000x reference)
import jax
import jax.numpy as jnp
from jax.experimental import pallas as pl
from jax.experimental.pallas import tpu as pltpu
from jax.experimental.pallas import tpu_sc as plsc

_BR = 16
_LANES = 16
_UNROLL = 4


def kernel(x, emb_weight):
    batch, num_patches, dim = x.shape
    rows = batch * num_patches
    nbe = num_patches // _BR
    x2 = x.reshape(rows, dim)

    mesh = plsc.VectorSubcoreMesh(core_axis_name="c", subcore_axis_name="s")

    @pl.kernel(out_type=jax.ShapeDtypeStruct((rows, dim), x.dtype), mesh=mesh)
    def sc_kernel(x_hbm, emb_hbm, o_hbm):
        def body(x_vmem, emb_vmem, o_vmem):
            @pl.loop(0, _BR)
            def _(r):
                @pl.loop(0, dim, step=_LANES * _UNROLL)
                def _(c):
                    for u in range(_UNROLL):
                        cs = pl.ds(c + u * _LANES, _LANES)
                        o_vmem.at[pl.ds(r, 1), cs][...] = (
                            x_vmem.at[pl.ds(r, 1), cs][...]
                            + emb_vmem.at[pl.ds(r, 1), cs][...]
                        )

        pltpu.emit_pipeline(
            body,
            grid=(rows // _BR,),
            in_specs=[
                pl.BlockSpec((_BR, dim), lambda i: (i, 0)),
                pl.BlockSpec((_BR, dim), lambda i: (i % nbe, 0)),
            ],
            out_specs=[pl.BlockSpec((_BR, dim), lambda i: (i, 0))],
            core_axis_name=("c", "s"),
            dimension_semantics=(pltpu.PARALLEL,),
        )(x_hbm, emb_hbm, o_hbm)

    return sc_kernel(x2, emb_weight).reshape(x.shape)


# speedup vs baseline: 4.3945x; 4.3945x over previous
import jax
import jax.numpy as jnp
from jax.experimental import pallas as pl
from jax.experimental.pallas import tpu as pltpu

_BR = 2048


def _add_body(x_ref, emb_ref, out_ref):
    out_ref[0] = x_ref[0] + emb_ref[...]


def kernel(x, emb_weight):
    batch, num_patches, dim = x.shape
    nb = num_patches // _BR
    return pl.pallas_call(
        _add_body,
        grid=(nb, batch),
        in_specs=[
            pl.BlockSpec((1, _BR, dim), lambda i, b: (b, i, 0)),
            pl.BlockSpec((_BR, dim), lambda i, b: (i, 0)),
        ],
        out_specs=pl.BlockSpec((1, _BR, dim), lambda i, b: (b, i, 0)),
        out_shape=jax.ShapeDtypeStruct(x.shape, x.dtype),
        compiler_params=pltpu.CompilerParams(
            dimension_semantics=("parallel", "arbitrary"),
        ),
    )(x, emb_weight)
